# Initial kernel scaffold; baseline (speedup 1.0000x reference)
import jax
import jax.numpy as jnp
from jax.experimental import pallas as pl


def kernel(x, edge_index, edge_type, rel_embed, weight, rel_weight, bias):
    raise NotImplementedError("write your pallas kernel here")


# speedup vs baseline: 3.4544x; 3.4544x over previous
import functools

import jax
import jax.numpy as jnp
from jax import lax
from jax.experimental import pallas as pl
from jax.experimental.pallas import tpu as pltpu
from jax.experimental.pallas import tpu_sc as plsc

N = 10000
D = 128
NUM_REL_ROWS = 200
TBL_ROWS = N + NUM_REL_ROWS + 8
ZROW = N + NUM_REL_ROWS

NC, NS = 2, 16
CH = 128
T_CHUNKS = 160
SB = 32
TOT_CHUNKS = NC * NS * T_CHUNKS
TOT_SLOTS = TOT_CHUNKS * CH
ACC_ROWS = 10240
ROWS_PER_TILE = ACC_ROWS // NS
DUMMY_DST = N


def _sc_scatter(table, gidx, didx, zeros640):
    mesh = plsc.VectorSubcoreMesh(
        core_axis_name="c", subcore_axis_name="s", num_cores=NC, num_subcores=NS)

    @functools.partial(
        pl.kernel,
        out_type=jax.ShapeDtypeStruct((NC, ACC_ROWS, D), jnp.float32),
        mesh=mesh,
        scratch_types=[
            pltpu.VMEM((SB, CH), jnp.int32),
            pltpu.VMEM((SB, CH), jnp.int32),
            pltpu.VMEM((CH, D), jnp.float32),
            pltpu.VMEM_SHARED((ACC_ROWS, D), jnp.float32),
            pltpu.SemaphoreType.DMA,
        ],
    )
    def k(table_hbm, gidx_hbm, didx_hbm, z_hbm, out_hbm, gv, dv, rows, acc, sem):
        c = lax.axis_index("c")
        s = lax.axis_index("s")
        wid = c * NS + s
        base = wid * T_CHUNKS

        pltpu.sync_copy(z_hbm, acc.at[pl.ds(s * ROWS_PER_TILE, ROWS_PER_TILE)])
        plsc.subcore_barrier()

        @pl.loop(0, T_CHUNKS // SB)
        def _stage(j):
            pltpu.sync_copy(gidx_hbm.at[pl.ds(base + j * SB, SB)], gv)
            pltpu.sync_copy(didx_hbm.at[pl.ds(base + j * SB, SB)], dv)

            @pl.loop(0, SB)
            def _chunk(i):
                pltpu.async_copy(table_hbm.at[gv.at[i]], rows, sem).wait()
                pltpu.sync_copy(rows, acc.at[dv.at[i]], add=True)

        plsc.subcore_barrier()
        pltpu.sync_copy(
            acc.at[pl.ds(s * ROWS_PER_TILE, ROWS_PER_TILE)],
            out_hbm.at[c, pl.ds(s * ROWS_PER_TILE, ROWS_PER_TILE)])

    return k(table, gidx, didx, zeros640)


def _tc_out_matmul(x, acc, weight, bias2d):
    BM = 1000

    def body(x_ref, a0_ref, a1_ref, w_ref, b_ref, o_ref):
        s = x_ref[...] + a0_ref[0] + a1_ref[0]
        o_ref[...] = jnp.dot(s, w_ref[...],
                             preferred_element_type=jnp.float32) + b_ref[...]

    return pl.pallas_call(
        body,
        grid=(N // BM,),
        in_specs=[
            pl.BlockSpec((BM, D), lambda i: (i, 0)),
            pl.BlockSpec((1, BM, D), lambda i: (0, i, 0)),
            pl.BlockSpec((1, BM, D), lambda i: (1, i, 0)),
            pl.BlockSpec((D, D), lambda i: (0, 0)),
            pl.BlockSpec((1, D), lambda i: (0, 0)),
        ],
        out_specs=pl.BlockSpec((BM, D), lambda i: (i, 0)),
        out_shape=jax.ShapeDtypeStruct((N, D), jnp.float32),
    )(x, acc, acc, weight, bias2d)


def _tc_rel_matmul(rel_pad, rel_weight):
    def body(r_ref, w_ref, o_ref):
        o_ref[...] = jnp.dot(r_ref[...], w_ref[...],
                             preferred_element_type=jnp.float32)

    return pl.pallas_call(
        body,
        out_shape=jax.ShapeDtypeStruct((rel_pad.shape[0], D), jnp.float32),
    )(rel_pad, rel_weight)


def kernel(x, edge_index, edge_type, rel_embed, weight, rel_weight, bias):
    E = edge_index.shape[1]
    src = edge_index[0]
    dst = edge_index[1]

    table = jnp.concatenate(
        [x, -rel_embed, jnp.zeros((TBL_ROWS - N - NUM_REL_ROWS, D), jnp.float32)])

    gidx = jnp.concatenate([src, edge_type + N])
    didx = jnp.concatenate([dst, dst])
    pad = TOT_SLOTS - 2 * E
    gidx = jnp.concatenate([gidx, jnp.full((pad,), ZROW, jnp.int32)])
    didx = jnp.concatenate([didx, jnp.full((pad,), DUMMY_DST, jnp.int32)])
    gidx = gidx.reshape(TOT_CHUNKS, CH)
    didx = didx.reshape(TOT_CHUNKS, CH)

    zeros640 = jnp.zeros((ROWS_PER_TILE, D), jnp.float32)
    acc = _sc_scatter(table, gidx, didx, zeros640)

    out = _tc_out_matmul(x, acc, weight, bias.reshape(1, D))

    rel_pad = jnp.concatenate([rel_embed, jnp.zeros((8, D), jnp.float32)])
    rel_out = _tc_rel_matmul(rel_pad, rel_weight)[:NUM_REL_ROWS + 1]
    return (out, rel_out)


# speedup vs baseline: 10.1478x; 2.9377x over previous
import functools

import jax
import jax.numpy as jnp
from jax import lax
from jax.experimental import pallas as pl
from jax.experimental.pallas import tpu as pltpu
from jax.experimental.pallas import tpu_sc as plsc

N = 10000
D = 128
NUM_REL_ROWS = 200
TBL_ROWS = N + NUM_REL_ROWS + 8
ZROW = N + NUM_REL_ROWS

NC, NS = 2, 16
CH = 128
T_CHUNKS = 160
SB = 40
TOT_CHUNKS = NC * NS * T_CHUNKS
TOT_SLOTS = TOT_CHUNKS * CH
ACC_ROWS = 10240
ROWS_PER_TILE = ACC_ROWS // NS
ZB = 64


def _sc_scatter(table, gidx, didx, zblk):
    mesh = plsc.VectorSubcoreMesh(
        core_axis_name="c", subcore_axis_name="s", num_cores=NC, num_subcores=NS)

    @functools.partial(
        pl.kernel,
        out_type=jax.ShapeDtypeStruct((NC, ACC_ROWS, D), jnp.float32),
        mesh=mesh,
        scratch_types=[
            pltpu.VMEM((SB, CH), jnp.int32),
            pltpu.VMEM((SB, CH), jnp.int32),
            pltpu.VMEM((CH, D), jnp.float32),
            pltpu.VMEM((CH, D), jnp.float32),
            pltpu.VMEM_SHARED((ACC_ROWS, D), jnp.float32),
            pltpu.SemaphoreType.DMA,
            pltpu.SemaphoreType.DMA,
        ],
    )
    def k(table_hbm, gidx_hbm, didx_hbm, z_hbm, out_hbm,
          gv, dv, rows0, rows1, acc, sem0, sem1):
        c = lax.axis_index("c")
        s = lax.axis_index("s")
        wid = c * NS + s
        base = wid * T_CHUNKS

        pltpu.sync_copy(z_hbm, acc.at[pl.ds(s * ROWS_PER_TILE, ROWS_PER_TILE)])
        plsc.subcore_barrier()

        rows = (rows0, rows1)
        sems = (sem0, sem1)

        @pl.loop(0, T_CHUNKS // SB)
        def _stage(j):
            pltpu.sync_copy(gidx_hbm.at[pl.ds(base + j * SB, SB)], gv)
            pltpu.sync_copy(didx_hbm.at[pl.ds(base + j * SB, SB)], dv)
            pltpu.async_copy(table_hbm.at[gv.at[0]], rows0, sem0)
            pltpu.async_copy(table_hbm.at[gv.at[1]], rows1, sem1)

            @pl.loop(0, SB // 2)
            def _pair(m):
                for b in range(2):
                    local = 2 * m + b
                    pltpu.make_async_copy(table_hbm.at[gv.at[local]],
                                          rows[b], sems[b]).wait()
                    pltpu.sync_copy(rows[b], acc.at[dv.at[local]], add=True)

                    @pl.when(local + 2 < SB)
                    def _():
                        pltpu.async_copy(table_hbm.at[gv.at[local + 2]],
                                         rows[b], sems[b])

        plsc.subcore_barrier()
        pltpu.sync_copy(
            acc.at[pl.ds(s * ROWS_PER_TILE, ROWS_PER_TILE)],
            out_hbm.at[c, pl.ds(s * ROWS_PER_TILE, ROWS_PER_TILE)])

    return k(table, gidx, didx, zblk)


def _tc_out_matmul(x, acc, weight, bias2d):
    BM = 1000

    def body(x_ref, a0_ref, a1_ref, w_ref, b_ref, o_ref):
        s = x_ref[...] + a0_ref[0] + a1_ref[0]
        o_ref[...] = jnp.dot(s, w_ref[...],
                             preferred_element_type=jnp.float32) + b_ref[...]

    return pl.pallas_call(
        body,
        grid=(N // BM,),
        in_specs=[
            pl.BlockSpec((BM, D), lambda i: (i, 0)),
            pl.BlockSpec((1, BM, D), lambda i: (0, i, 0)),
            pl.BlockSpec((1, BM, D), lambda i: (1, i, 0)),
            pl.BlockSpec((D, D), lambda i: (0, 0)),
            pl.BlockSpec((1, D), lambda i: (0, 0)),
        ],
        out_specs=pl.BlockSpec((BM, D), lambda i: (i, 0)),
        out_shape=jax.ShapeDtypeStruct((N, D), jnp.float32),
    )(x, acc, acc, weight, bias2d)


def _tc_rel_matmul(rel_pad, rel_weight):
    def body(r_ref, w_ref, o_ref):
        o_ref[...] = jnp.dot(r_ref[...], w_ref[...],
                             preferred_element_type=jnp.float32)

    return pl.pallas_call(
        body,
        out_shape=jax.ShapeDtypeStruct((rel_pad.shape[0], D), jnp.float32),
    )(rel_pad, rel_weight)


def kernel(x, edge_index, edge_type, rel_embed, weight, rel_weight, bias):
    E = edge_index.shape[1]
    src = edge_index[0]
    dst = edge_index[1]

    table = jnp.concatenate(
        [x, -rel_embed, jnp.zeros((TBL_ROWS - N - NUM_REL_ROWS, D), jnp.float32)])

    gidx = jnp.concatenate([src, edge_type + N])
    didx = jnp.concatenate([dst, dst])
    pad = TOT_SLOTS - 2 * E
    pad_ar = jnp.arange(pad, dtype=jnp.int32)
    gidx = jnp.concatenate([gidx, ZROW + (pad_ar % 8)])
    didx = jnp.concatenate([didx, N + (pad_ar % (ACC_ROWS - N))])
    gidx = gidx.reshape(TOT_CHUNKS, CH)
    didx = didx.reshape(TOT_CHUNKS, CH)

    zblk = jnp.zeros((ROWS_PER_TILE, D), jnp.float32)
    acc = _sc_scatter(table, gidx, didx, zblk)

    out = _tc_out_matmul(x, acc, weight, bias.reshape(1, D))

    rel_pad = jnp.concatenate([rel_embed, jnp.zeros((8, D), jnp.float32)])
    rel_out = _tc_rel_matmul(rel_pad, rel_weight)[:NUM_REL_ROWS + 1]
    return (out, rel_out)


# speedup vs baseline: 10.5518x; 1.0398x over previous
import functools

import jax
import jax.numpy as jnp
from jax import lax
from jax.experimental import pallas as pl
from jax.experimental.pallas import tpu as pltpu
from jax.experimental.pallas import tpu_sc as plsc

N = 10000
D = 128
NUM_REL_ROWS = 200
REL_ROWS = NUM_REL_ROWS + 8
ZROW = NUM_REL_ROWS

NC, NS = 2, 16
CH = 128
T_CHUNKS = 160
SB = 40
TOT_CHUNKS = NC * NS * T_CHUNKS
TOT_SLOTS = TOT_CHUNKS * CH
NSRC_CHUNKS = 2500
ACC_ROWS = 10240
ROWS_PER_TILE = ACC_ROWS // NS


def _sc_scatter(x, negrel, gidx, didx, zblk):
    mesh = plsc.VectorSubcoreMesh(
        core_axis_name="c", subcore_axis_name="s", num_cores=NC, num_subcores=NS)

    @functools.partial(
        pl.kernel,
        out_type=jax.ShapeDtypeStruct((NC, ACC_ROWS, D), jnp.float32),
        mesh=mesh,
        scratch_types=[
            pltpu.VMEM((SB, CH), jnp.int32),
            pltpu.VMEM((SB, CH), jnp.int32),
            pltpu.VMEM((CH, D), jnp.float32),
            pltpu.VMEM((CH, D), jnp.float32),
            pltpu.VMEM_SHARED((ACC_ROWS, D), jnp.float32),
            pltpu.VMEM_SHARED((REL_ROWS, D), jnp.float32),
            pltpu.SemaphoreType.DMA,
            pltpu.SemaphoreType.DMA,
        ],
    )
    def k(x_hbm, negrel_hbm, gidx_hbm, didx_hbm, z_hbm, out_hbm,
          gv, dv, rows0, rows1, acc, nrel, sem0, sem1):
        c = lax.axis_index("c")
        s = lax.axis_index("s")
        wid = c * NS + s
        base = wid * T_CHUNKS

        pltpu.sync_copy(z_hbm, acc.at[pl.ds(s * ROWS_PER_TILE, ROWS_PER_TILE)])

        @pl.when(s == 0)
        def _():
            pltpu.sync_copy(negrel_hbm, nrel)

        plsc.subcore_barrier()

        rows = (rows0, rows1)
        sems = (sem0, sem1)

        def issue_gather(stage_base, local, b):
            chunk = stage_base + local

            @pl.when(chunk < NSRC_CHUNKS)
            def _():
                pltpu.async_copy(x_hbm.at[gv.at[local]], rows[b], sems[b])

            @pl.when(chunk >= NSRC_CHUNKS)
            def _():
                pltpu.async_copy(nrel.at[gv.at[local]], rows[b], sems[b])

        @pl.loop(0, T_CHUNKS // SB)
        def _stage(j):
            stage_base = base + j * SB
            pltpu.sync_copy(gidx_hbm.at[pl.ds(stage_base, SB)], gv)
            pltpu.sync_copy(didx_hbm.at[pl.ds(stage_base, SB)], dv)
            issue_gather(stage_base, 0, 0)
            issue_gather(stage_base, 1, 1)

            @pl.loop(0, SB // 2)
            def _pair(m):
                for b in range(2):
                    local = 2 * m + b
                    pltpu.make_async_copy(
                        x_hbm.at[pl.ds(0, CH)], rows[b], sems[b]).wait()
                    pltpu.sync_copy(rows[b], acc.at[dv.at[local]], add=True)

                    @pl.when(local + 2 < SB)
                    def _():
                        issue_gather(stage_base, local + 2, b)

        plsc.subcore_barrier()
        pltpu.sync_copy(
            acc.at[pl.ds(s * ROWS_PER_TILE, ROWS_PER_TILE)],
            out_hbm.at[c, pl.ds(s * ROWS_PER_TILE, ROWS_PER_TILE)])

    return k(x, negrel, gidx, didx, zblk)


def _tc_out_matmul(x, acc, weight, bias2d):
    BM = 1000

    def body(x_ref, a0_ref, a1_ref, w_ref, b_ref, o_ref):
        s = x_ref[...] + a0_ref[0] + a1_ref[0]
        o_ref[...] = jnp.dot(s, w_ref[...],
                             preferred_element_type=jnp.float32) + b_ref[...]

    return pl.pallas_call(
        body,
        grid=(N // BM,),
        in_specs=[
            pl.BlockSpec((BM, D), lambda i: (i, 0)),
            pl.BlockSpec((1, BM, D), lambda i: (0, i, 0)),
            pl.BlockSpec((1, BM, D), lambda i: (1, i, 0)),
            pl.BlockSpec((D, D), lambda i: (0, 0)),
            pl.BlockSpec((1, D), lambda i: (0, 0)),
        ],
        out_specs=pl.BlockSpec((BM, D), lambda i: (i, 0)),
        out_shape=jax.ShapeDtypeStruct((N, D), jnp.float32),
    )(x, acc, acc, weight, bias2d)


def _tc_rel_matmul(rel_pad, rel_weight):
    def body(r_ref, w_ref, o_ref):
        o_ref[...] = jnp.dot(r_ref[...], w_ref[...],
                             preferred_element_type=jnp.float32)

    return pl.pallas_call(
        body,
        out_shape=jax.ShapeDtypeStruct((rel_pad.shape[0], D), jnp.float32),
    )(rel_pad, rel_weight)


def kernel(x, edge_index, edge_type, rel_embed, weight, rel_weight, bias):
    E = edge_index.shape[1]
    src = edge_index[0]
    dst = edge_index[1]

    negrel = jnp.concatenate(
        [-rel_embed, jnp.zeros((REL_ROWS - NUM_REL_ROWS, D), jnp.float32)])

    pad = TOT_SLOTS - 2 * E
    pad_ar = jnp.arange(pad, dtype=jnp.int32)
    gidx = jnp.concatenate([src, edge_type, ZROW + (pad_ar % 8)])
    didx = jnp.concatenate([dst, dst, N + (pad_ar % (ACC_ROWS - N))])
    gidx = gidx.reshape(TOT_CHUNKS, CH)
    didx = didx.reshape(TOT_CHUNKS, CH)

    zblk = jnp.zeros((ROWS_PER_TILE, D), jnp.float32)
    acc = _sc_scatter(x, negrel, gidx, didx, zblk)

    out = _tc_out_matmul(x, acc, weight, bias.reshape(1, D))

    rel_pad = jnp.concatenate([rel_embed, jnp.zeros((8, D), jnp.float32)])
    rel_out = _tc_rel_matmul(rel_pad, rel_weight)[:NUM_REL_ROWS + 1]
    return (out, rel_out)
